# baseline (speedup 1.0000x reference)
import jax
import jax.numpy as jnp
from jax.experimental import pallas as pl
from jax.experimental.pallas import tpu as pltpu

_HZ_TO_K = 6.62607015e-34 / 1.380649e-23


def _body(scale_ref, down_ref, up_ref, e_ref, o_ref):
    e = jnp.exp(e_ref[...] * scale_ref[0, 0])
    denom = jnp.sum(e, axis=-1)
    iota = jax.lax.broadcasted_iota(jnp.int32, down_ref.shape + (e.shape[-1],), 2)
    msk = ((iota == down_ref[...][:, :, None]).astype(jnp.float32)
           - (iota == up_ref[...][:, :, None]).astype(jnp.float32))
    num = jnp.sum(e * msk, axis=-1)
    o_ref[...] = num / denom


def kernel(energies, lvl_down, lvl_up, temperature):
    B, M, L = energies.shape
    scale = (-_HZ_TO_K / temperature.astype(jnp.float32)).reshape(1, 1)
    down = lvl_down.astype(jnp.int32).reshape(1, M)
    up = lvl_up.astype(jnp.int32).reshape(1, M)

    B_BLK = 128
    grid = (B // B_BLK,)
    return pl.pallas_call(
        _body,
        grid=grid,
        in_specs=[
            pl.BlockSpec(memory_space=pltpu.SMEM),
            pl.BlockSpec((1, M), lambda i: (0, 0)),
            pl.BlockSpec((1, M), lambda i: (0, 0)),
            pl.BlockSpec((B_BLK, M, L), lambda i: (i, 0, 0)),
        ],
        out_specs=pl.BlockSpec((B_BLK, M), lambda i: (i, 0)),
        out_shape=jax.ShapeDtypeStruct((B, M), jnp.float32),
        compiler_params=pltpu.CompilerParams(
            dimension_semantics=("arbitrary",)),
    )(scale, down, up, energies)


# speedup vs baseline: 4.2247x; 4.2247x over previous
import functools

import jax
import jax.numpy as jnp
from jax import lax
from jax.experimental import pallas as pl
from jax.experimental.pallas import tpu as pltpu
from jax.experimental.pallas import tpu_sc as plsc

_HZ_TO_K = 6.62607015e-34 / 1.380649e-23
_LANES = 16


def _den_body(scale_ref, e_ref, o_ref):
    e = jnp.exp(e_ref[...] * scale_ref[0, 0])
    o_ref[...] = jnp.sum(e, axis=1)


def _den_kernel(eT, scale, M, L, B):
    MB, BC = 8, 2048
    grid = (M // MB, B // BC)
    return pl.pallas_call(
        _den_body,
        grid=grid,
        in_specs=[
            pl.BlockSpec(memory_space=pltpu.SMEM),
            pl.BlockSpec((MB, L, BC), lambda i, j: (i, 0, j)),
        ],
        out_specs=pl.BlockSpec((MB, BC), lambda i, j: (i, j)),
        out_shape=jax.ShapeDtypeStruct((M, B), jnp.float32),
        compiler_params=pltpu.CompilerParams(
            dimension_semantics=("arbitrary", "arbitrary")),
    )(scale, eT)


def _make_num_kernel(M, L, B):
    nq = 4
    bc = B // nq
    dq = bc // 128
    units = M * nq
    nw = 32
    upw = units // nw
    assert upw * nw == units and upw % 2 == 1
    mesh = plsc.VectorSubcoreMesh(core_axis_name="c", subcore_axis_name="s")

    @functools.partial(
        pl.kernel,
        mesh=mesh,
        compiler_params=pltpu.CompilerParams(needs_layout_passes=False),
        out_type=jax.ShapeDtypeStruct((M, B), jnp.float32),
        scratch_types=[
            pltpu.VMEM((dq, 128), jnp.float32),
            pltpu.VMEM((dq, 128), jnp.float32),
            pltpu.VMEM((dq, 128), jnp.float32),
            pltpu.VMEM((dq, 128), jnp.float32),
            pltpu.VMEM((bc,), jnp.float32),
            pltpu.VMEM((bc,), jnp.float32),
            pltpu.VMEM((bc,), jnp.float32),
            pltpu.VMEM((bc,), jnp.float32),
            pltpu.VMEM((256,), jnp.int32),
            pltpu.VMEM((256,), jnp.int32),
            pltpu.VMEM((_LANES,), jnp.float32),
            pltpu.SemaphoreType.DMA,
            pltpu.SemaphoreType.DMA,
            pltpu.SemaphoreType.DMA,
        ],
    )
    def k(e4_hbm, den_hbm, rows_dn_hbm, rows_up_hbm, scale_hbm, out_hbm,
          xd_a, xu_a, xd_b, xu_b, dv_a, dv_b, ov_a, ov_b,
          rdn_v, rup_v, scale_v, sem_in, sem_oa, sem_ob):
        wid = lax.axis_index("s") * 2 + lax.axis_index("c")

        pltpu.sync_copy(rows_dn_hbm, rdn_v)
        pltpu.sync_copy(rows_up_hbm, rup_v)
        pltpu.sync_copy(scale_hbm, scale_v)
        s = scale_v[...]

        def unit_of(i):
            u = wid * upw + i
            return u // nq, lax.rem(u, nq)

        def row_scalar(tab, m):
            mv = lax.broadcast_in_dim(m, (_LANES,), ())
            return jnp.max(plsc.load_gather(tab, [mv]))

        def in_copies(i, xd, xu, dv):
            m, q = unit_of(i)
            rd = row_scalar(rdn_v, m)
            ru = row_scalar(rup_v, m)
            return (
                pltpu.make_async_copy(
                    e4_hbm.at[rd // 8, pl.ds(q * dq, dq), lax.rem(rd, 8)],
                    xd, sem_in),
                pltpu.make_async_copy(
                    e4_hbm.at[ru // 8, pl.ds(q * dq, dq), lax.rem(ru, 8)],
                    xu, sem_in),
                pltpu.make_async_copy(
                    den_hbm.at[m, pl.ds(q * bc, bc)], dv, sem_in),
            )

        def start_in(i, xd, xu, dv):
            for c in in_copies(i, xd, xu, dv):
                c.start()

        def wait_in(i, xd, xu, dv):
            for c in in_copies(i, xd, xu, dv):
                c.wait()

        def out_copy(i, ov, sem):
            m, q = unit_of(i)
            return pltpu.make_async_copy(
                ov, out_hbm.at[m, pl.ds(q * bc, bc)], sem)

        def compute(xd, xu, dv, ov):
            for j in range(bc // _LANES):
                dd, e0 = j // 8, (j % 8) * _LANES
                ed = jnp.exp(xd[dd, pl.ds(e0, _LANES)] * s)
                eu = jnp.exp(xu[dd, pl.ds(e0, _LANES)] * s)
                den = dv[pl.ds(j * _LANES, _LANES)]
                ov[pl.ds(j * _LANES, _LANES)] = (ed - eu) / den

        start_in(0, xd_a, xu_a, dv_a)

        def body(i, _):
            ua = 2 * i
            ub = 2 * i + 1
            wait_in(ua, xd_a, xu_a, dv_a)
            start_in(ub, xd_b, xu_b, dv_b)

            @pl.when(i > 0)
            def _():
                out_copy(ua - 2, ov_a, sem_oa).wait()

            compute(xd_a, xu_a, dv_a, ov_a)
            out_copy(ua, ov_a, sem_oa).start()

            wait_in(ub, xd_b, xu_b, dv_b)

            @pl.when(ub + 1 < upw)
            def _():
                start_in(ub + 1, xd_a, xu_a, dv_a)

            @pl.when(i > 0)
            def _():
                out_copy(ub - 2, ov_b, sem_ob).wait()

            compute(xd_b, xu_b, dv_b, ov_b)
            out_copy(ub, ov_b, sem_ob).start()
            return 0

        lax.fori_loop(0, upw // 2, body, 0)
        wait_in(upw - 1, xd_a, xu_a, dv_a)
        out_copy(upw - 3, ov_a, sem_oa).wait()
        compute(xd_a, xu_a, dv_a, ov_a)
        out_copy(upw - 1, ov_a, sem_oa).start()
        out_copy(upw - 2, ov_b, sem_ob).wait()
        out_copy(upw - 1, ov_a, sem_oa).wait()

    return k


def kernel(energies, lvl_down, lvl_up, temperature):
    B, M, L = energies.shape
    scale = -_HZ_TO_K / temperature.astype(jnp.float32)
    eT = jnp.transpose(energies, (1, 2, 0))
    e4 = (eT.reshape(M * L // 8, 8, B // 128, 128)
            .transpose(0, 2, 1, 3))
    den = _den_kernel(eT, scale.reshape(1, 1), M, L, B)

    mm = jnp.arange(M, dtype=jnp.int32)
    rows_dn = jnp.pad(mm * L + lvl_down.astype(jnp.int32), (0, 256 - M))
    rows_up = jnp.pad(mm * L + lvl_up.astype(jnp.int32), (0, 256 - M))
    scale_v = jnp.broadcast_to(scale, (_LANES,))

    out_t = _make_num_kernel(M, L, B)(e4, den, rows_dn, rows_up, scale_v)
    return out_t.T


# speedup vs baseline: 4.4870x; 1.0621x over previous
import functools

import jax
import jax.numpy as jnp
from jax import lax
from jax.experimental import pallas as pl
from jax.experimental.pallas import tpu as pltpu
from jax.experimental.pallas import tpu_sc as plsc

_HZ_TO_K = 6.62607015e-34 / 1.380649e-23
_LANES = 16


def _den_body(scale_ref, e_ref, o_ref):
    e = jnp.exp(e_ref[...] * scale_ref[0, 0])
    o_ref[...] = jnp.sum(e, axis=1)


def _den_kernel(eT, scale, M, L, B):
    MB, BC = 8, 2048
    grid = (M // MB, B // BC)
    return pl.pallas_call(
        _den_body,
        grid=grid,
        in_specs=[
            pl.BlockSpec(memory_space=pltpu.SMEM),
            pl.BlockSpec((MB, L, BC), lambda i, j: (i, 0, j)),
        ],
        out_specs=pl.BlockSpec((MB, BC), lambda i, j: (i, j)),
        out_shape=jax.ShapeDtypeStruct((M, B), jnp.float32),
        compiler_params=pltpu.CompilerParams(
            dimension_semantics=("arbitrary", "arbitrary")),
    )(scale, eT)


def _make_num_kernel(M, L, B):
    nq = 4
    bc = B // nq
    dq = bc // 128
    units = M * nq
    nw = 32
    upw = units // nw
    assert upw * nw == units and upw % 2 == 1
    mesh = plsc.VectorSubcoreMesh(core_axis_name="c", subcore_axis_name="s")

    @functools.partial(
        pl.kernel,
        mesh=mesh,
        compiler_params=pltpu.CompilerParams(needs_layout_passes=False),
        out_type=jax.ShapeDtypeStruct((M, B), jnp.float32),
        scratch_types=[
            pltpu.VMEM((dq, 128), jnp.float32),
            pltpu.VMEM((dq, 128), jnp.float32),
            pltpu.VMEM((dq, 128), jnp.float32),
            pltpu.VMEM((dq, 128), jnp.float32),
            pltpu.VMEM((bc,), jnp.float32),
            pltpu.VMEM((bc,), jnp.float32),
            pltpu.VMEM((256,), jnp.int32),
            pltpu.VMEM((256,), jnp.int32),
            pltpu.VMEM((_LANES,), jnp.float32),
            pltpu.SemaphoreType.DMA,
            pltpu.SemaphoreType.DMA,
            pltpu.SemaphoreType.DMA,
        ],
    )
    def k(e4_hbm, rows_dn_hbm, rows_up_hbm, scale_hbm, out_hbm,
          xd_a, xu_a, xd_b, xu_b, ov_a, ov_b,
          rdn_v, rup_v, scale_v, sem_in, sem_oa, sem_ob):
        wid = lax.axis_index("s") * 2 + lax.axis_index("c")

        pltpu.sync_copy(rows_dn_hbm, rdn_v)
        pltpu.sync_copy(rows_up_hbm, rup_v)
        pltpu.sync_copy(scale_hbm, scale_v)
        s = scale_v[...]

        def unit_of(i):
            u = wid * upw + i
            return u // nq, lax.rem(u, nq)

        def row_scalar(tab, m):
            mv = lax.broadcast_in_dim(m, (_LANES,), ())
            return jnp.max(plsc.load_gather(tab, [mv]))

        def in_copies(i, xd, xu):
            m, q = unit_of(i)
            rd = row_scalar(rdn_v, m)
            ru = row_scalar(rup_v, m)
            return (
                pltpu.make_async_copy(
                    e4_hbm.at[rd // 8, pl.ds(q * dq, dq), lax.rem(rd, 8)],
                    xd, sem_in),
                pltpu.make_async_copy(
                    e4_hbm.at[ru // 8, pl.ds(q * dq, dq), lax.rem(ru, 8)],
                    xu, sem_in),
            )

        def start_in(i, xd, xu):
            for c in in_copies(i, xd, xu):
                c.start()

        def wait_in(i, xd, xu):
            for c in in_copies(i, xd, xu):
                c.wait()

        def out_copy(i, ov, sem):
            m, q = unit_of(i)
            return pltpu.make_async_copy(
                ov, out_hbm.at[m, pl.ds(q * bc, bc)], sem)

        def compute(xd, xu, ov):
            for j in range(bc // _LANES):
                dd, e0 = j // 8, (j % 8) * _LANES
                ed = jnp.exp(xd[dd, pl.ds(e0, _LANES)] * s)
                eu = jnp.exp(xu[dd, pl.ds(e0, _LANES)] * s)
                ov[pl.ds(j * _LANES, _LANES)] = ed - eu

        start_in(0, xd_a, xu_a)

        def body(i, _):
            ua = 2 * i
            ub = 2 * i + 1
            wait_in(ua, xd_a, xu_a)
            start_in(ub, xd_b, xu_b)

            @pl.when(i > 0)
            def _():
                out_copy(ua - 2, ov_a, sem_oa).wait()

            compute(xd_a, xu_a, ov_a)
            out_copy(ua, ov_a, sem_oa).start()

            wait_in(ub, xd_b, xu_b)

            @pl.when(ub + 1 < upw)
            def _():
                start_in(ub + 1, xd_a, xu_a)

            @pl.when(i > 0)
            def _():
                out_copy(ub - 2, ov_b, sem_ob).wait()

            compute(xd_b, xu_b, ov_b)
            out_copy(ub, ov_b, sem_ob).start()
            return 0

        lax.fori_loop(0, upw // 2, body, 0)
        wait_in(upw - 1, xd_a, xu_a)
        out_copy(upw - 3, ov_a, sem_oa).wait()
        compute(xd_a, xu_a, ov_a)
        out_copy(upw - 1, ov_a, sem_oa).start()
        out_copy(upw - 2, ov_b, sem_ob).wait()
        out_copy(upw - 1, ov_a, sem_oa).wait()

    return k


def _div_body(n_ref, d_ref, o_ref):
    o_ref[...] = n_ref[...] / d_ref[...]


def _div_kernel(num, den, M, B):
    MB = 8
    grid = (M // MB,)
    return pl.pallas_call(
        _div_body,
        grid=grid,
        in_specs=[
            pl.BlockSpec((MB, B), lambda i: (i, 0)),
            pl.BlockSpec((MB, B), lambda i: (i, 0)),
        ],
        out_specs=pl.BlockSpec((MB, B), lambda i: (i, 0)),
        out_shape=jax.ShapeDtypeStruct((M, B), jnp.float32),
        compiler_params=pltpu.CompilerParams(
            dimension_semantics=("arbitrary",)),
    )(num, den)


def kernel(energies, lvl_down, lvl_up, temperature):
    B, M, L = energies.shape
    scale = -_HZ_TO_K / temperature.astype(jnp.float32)
    eT = jnp.transpose(energies, (1, 2, 0))
    e4 = (eT.reshape(M * L // 8, 8, B // 128, 128)
            .transpose(0, 2, 1, 3))
    mm = jnp.arange(M, dtype=jnp.int32)
    rows_dn = jnp.pad(mm * L + lvl_down.astype(jnp.int32), (0, 256 - M))
    rows_up = jnp.pad(mm * L + lvl_up.astype(jnp.int32), (0, 256 - M))
    scale_v = jnp.broadcast_to(scale, (_LANES,))

    num = _make_num_kernel(M, L, B)(e4, rows_dn, rows_up, scale_v)
    den = _den_kernel(eT, scale.reshape(1, 1), M, L, B)
    return _div_kernel(num, den, M, B).T


# speedup vs baseline: 5.4721x; 1.2195x over previous
import functools

import jax
import jax.numpy as jnp
from jax import lax
from jax.experimental import pallas as pl
from jax.experimental.pallas import tpu as pltpu
from jax.experimental.pallas import tpu_sc as plsc

_HZ_TO_K = 6.62607015e-34 / 1.380649e-23
_LANES = 16


def _den_body(scale_ref, e_ref, o_ref):
    e = jnp.exp(e_ref[...] * scale_ref[0, 0])
    o_ref[...] = jnp.sum(e, axis=1)


def _den_kernel(eT, scale, M, L, B):
    MB, BC = 8, 4096
    grid = (M // MB, B // BC)
    return pl.pallas_call(
        _den_body,
        grid=grid,
        in_specs=[
            pl.BlockSpec(memory_space=pltpu.SMEM),
            pl.BlockSpec((MB, L, BC), lambda i, j: (i, 0, j)),
        ],
        out_specs=pl.BlockSpec((MB, BC), lambda i, j: (i, j)),
        out_shape=jax.ShapeDtypeStruct((M, B), jnp.float32),
        compiler_params=pltpu.CompilerParams(
            dimension_semantics=("arbitrary", "arbitrary")),
    )(scale, eT)


def _make_num_kernel(M, L, B):
    nq = 4
    bc = B // nq
    dq = bc // 128
    units = M * nq
    nw = 32
    upw = units // nw
    assert upw * nw == units and upw % 2 == 1
    mesh = plsc.VectorSubcoreMesh(core_axis_name="c", subcore_axis_name="s")

    @functools.partial(
        pl.kernel,
        mesh=mesh,
        compiler_params=pltpu.CompilerParams(needs_layout_passes=False),
        out_type=jax.ShapeDtypeStruct((M, B), jnp.float32),
        scratch_types=[
            pltpu.VMEM((dq, 128), jnp.float32),
            pltpu.VMEM((dq, 128), jnp.float32),
            pltpu.VMEM((dq, 128), jnp.float32),
            pltpu.VMEM((dq, 128), jnp.float32),
            pltpu.VMEM((bc,), jnp.float32),
            pltpu.VMEM((bc,), jnp.float32),
            pltpu.VMEM((256,), jnp.int32),
            pltpu.VMEM((256,), jnp.int32),
            pltpu.VMEM((_LANES,), jnp.float32),
            pltpu.SemaphoreType.DMA,
            pltpu.SemaphoreType.DMA,
            pltpu.SemaphoreType.DMA,
        ],
    )
    def k(e4_hbm, rows_dn_hbm, rows_up_hbm, scale_hbm, out_hbm,
          xd_a, xu_a, xd_b, xu_b, ov_a, ov_b,
          rdn_v, rup_v, scale_v, sem_in, sem_oa, sem_ob):
        wid = lax.axis_index("s") * 2 + lax.axis_index("c")

        pltpu.sync_copy(rows_dn_hbm, rdn_v)
        pltpu.sync_copy(rows_up_hbm, rup_v)
        pltpu.sync_copy(scale_hbm, scale_v)
        s = scale_v[...]

        def unit_of(i):
            u = wid * upw + i
            return u // nq, lax.rem(u, nq)

        def row_scalar(tab, m):
            mv = lax.broadcast_in_dim(m, (_LANES,), ())
            return jnp.max(plsc.load_gather(tab, [mv]))

        def in_copies(i, xd, xu):
            m, q = unit_of(i)
            rd = row_scalar(rdn_v, m)
            ru = row_scalar(rup_v, m)
            return (
                pltpu.make_async_copy(
                    e4_hbm.at[rd // 8, pl.ds(q * dq, dq), lax.rem(rd, 8)],
                    xd, sem_in),
                pltpu.make_async_copy(
                    e4_hbm.at[ru // 8, pl.ds(q * dq, dq), lax.rem(ru, 8)],
                    xu, sem_in),
            )

        def start_in(i, xd, xu):
            for c in in_copies(i, xd, xu):
                c.start()

        def wait_in(i, xd, xu):
            for c in in_copies(i, xd, xu):
                c.wait()

        def out_copy(i, ov, sem):
            m, q = unit_of(i)
            return pltpu.make_async_copy(
                ov, out_hbm.at[m, pl.ds(q * bc, bc)], sem)

        def compute(xd, xu, ov):
            for j in range(bc // _LANES):
                dd, e0 = j // 8, (j % 8) * _LANES
                ed = jnp.exp(xd[dd, pl.ds(e0, _LANES)] * s)
                eu = jnp.exp(xu[dd, pl.ds(e0, _LANES)] * s)
                ov[pl.ds(j * _LANES, _LANES)] = ed - eu

        start_in(0, xd_a, xu_a)

        def body(i, _):
            ua = 2 * i
            ub = 2 * i + 1
            wait_in(ua, xd_a, xu_a)
            start_in(ub, xd_b, xu_b)

            @pl.when(i > 0)
            def _():
                out_copy(ua - 2, ov_a, sem_oa).wait()

            compute(xd_a, xu_a, ov_a)
            out_copy(ua, ov_a, sem_oa).start()

            wait_in(ub, xd_b, xu_b)

            @pl.when(ub + 1 < upw)
            def _():
                start_in(ub + 1, xd_a, xu_a)

            @pl.when(i > 0)
            def _():
                out_copy(ub - 2, ov_b, sem_ob).wait()

            compute(xd_b, xu_b, ov_b)
            out_copy(ub, ov_b, sem_ob).start()
            return 0

        lax.fori_loop(0, upw // 2, body, 0)
        wait_in(upw - 1, xd_a, xu_a)
        out_copy(upw - 3, ov_a, sem_oa).wait()
        compute(xd_a, xu_a, ov_a)
        out_copy(upw - 1, ov_a, sem_oa).start()
        out_copy(upw - 2, ov_b, sem_ob).wait()
        out_copy(upw - 1, ov_a, sem_oa).wait()

    return k


def _div_body(n_ref, d_ref, o_ref):
    o_ref[...] = n_ref[...] / d_ref[...]


def _div_kernel(num, den, M, B):
    MB = 40
    grid = (M // MB,)
    return pl.pallas_call(
        _div_body,
        grid=grid,
        in_specs=[
            pl.BlockSpec((MB, B), lambda i: (i, 0)),
            pl.BlockSpec((MB, B), lambda i: (i, 0)),
        ],
        out_specs=pl.BlockSpec((MB, B), lambda i: (i, 0)),
        out_shape=jax.ShapeDtypeStruct((M, B), jnp.float32),
        compiler_params=pltpu.CompilerParams(
            dimension_semantics=("arbitrary",)),
    )(num, den)


def kernel(energies, lvl_down, lvl_up, temperature):
    B, M, L = energies.shape
    scale = -_HZ_TO_K / temperature.astype(jnp.float32)
    eT = jnp.transpose(energies, (1, 2, 0))
    e4 = (eT.reshape(M * L // 8, 8, B // 128, 128)
            .transpose(0, 2, 1, 3))
    mm = jnp.arange(M, dtype=jnp.int32)
    rows_dn = jnp.pad(mm * L + lvl_down.astype(jnp.int32), (0, 256 - M))
    rows_up = jnp.pad(mm * L + lvl_up.astype(jnp.int32), (0, 256 - M))
    scale_v = jnp.broadcast_to(scale, (_LANES,))

    num = _make_num_kernel(M, L, B)(e4, rows_dn, rows_up, scale_v)
    den = _den_kernel(eT, scale.reshape(1, 1), M, L, B)
    return _div_kernel(num, den, M, B).T
